# baseline (speedup 1.0000x reference)
import functools

import jax
import jax.numpy as jnp
from jax import lax
from jax.experimental import pallas as pl
from jax.experimental.pallas import tpu as pltpu
from jax.experimental.pallas import tpu_sc as plsc

N_NODES = 10000
N_EDGES = 320000
D = 128
BN_EPS = 1e-5
LEAKY_SLOPE = 0.01

NC = 2
NS = 16
NW = NC * NS
CHUNK = 112
CPB = 24
NBLK = 4
CPW = NBLK * CPB
E_PAD = NW * CPW * CHUNK
N_PAD = 10112
ROWS_PER_TILE = N_PAD // NS


def _sc_scatter_sum(h, src, dst, zinit):
    def body(h_hbm, src_hbm, dst_hbm, z_hbm, out_hbm,
             idx_s, idx_d, rows, sems, aggr):
        c = lax.axis_index("c")
        s = lax.axis_index("s")
        r0 = s * ROWS_PER_TILE
        pltpu.sync_copy(z_hbm.at[pl.ds(r0, ROWS_PER_TILE)],
                        aggr.at[pl.ds(r0, ROWS_PER_TILE)])
        plsc.subcore_barrier()

        def fire_g(t, buf):
            pltpu.async_copy(h_hbm.at[idx_s.at[t]], rows[buf], sems[buf])

        def wait_g(buf):
            pltpu.make_async_copy(
                h_hbm.at[idx_s.at[0]], rows[buf], sems[buf]).wait()

        def blk_body(b, carry0):
            b0 = pl.multiple_of(b * CPB, CPB)
            pltpu.sync_copy(src_hbm.at[c, s, pl.ds(b0, CPB)], idx_s)
            pltpu.sync_copy(dst_hbm.at[c, s, pl.ds(b0, CPB)], idx_d)
            fire_g(0, 0)
            fire_g(1, 1)

            def step(k, carry):
                t0 = 3 * k
                for u in range(3):
                    t = t0 + u
                    buf = u
                    wait_g(buf)
                    nxt = (u + 2) % 3
                    if u == 0:
                        fire_g(t + 2, nxt)
                    else:
                        @pl.when(k < CPB // 3 - 1)
                        def _(t=t, nxt=nxt):
                            fire_g(t + 2, nxt)
                    pltpu.sync_copy(rows[buf], aggr.at[idx_d.at[t]], add=True)
                return carry

            lax.fori_loop(0, CPB // 3, step, 0)
            return carry0

        lax.fori_loop(0, NBLK, blk_body, 0)
        plsc.subcore_barrier()
        pltpu.sync_copy(aggr.at[pl.ds(r0, ROWS_PER_TILE)],
                        out_hbm.at[c, pl.ds(r0, ROWS_PER_TILE)])

    mesh = plsc.VectorSubcoreMesh(core_axis_name="c", subcore_axis_name="s")
    run = pl.kernel(
        body,
        out_type=jax.ShapeDtypeStruct((NC, N_PAD, D), jnp.float32),
        mesh=mesh,
        scratch_types=[
            pltpu.VMEM((CPB, CHUNK), jnp.int32),
            pltpu.VMEM((CPB, CHUNK), jnp.int32),
            [pltpu.VMEM((CHUNK, D), jnp.float32) for _ in range(3)],
            [pltpu.SemaphoreType.DMA for _ in range(3)],
            pltpu.VMEM_SHARED((N_PAD, D), jnp.float32),
        ],
    )
    return run(h, src, dst, zinit)


def _tc_body(h_ref, p0_ref, p1_ref, w1_ref, b1_ref, w2_ref, b2_ref,
             g_ref, bt_ref, out_ref):
    h = h_ref[...]
    rst = h + p0_ref[...] + p1_ref[...]
    z = jnp.maximum(
        jnp.dot(rst, w1_ref[...], preferred_element_type=jnp.float32)
        + b1_ref[...], 0.0)
    z = jnp.dot(z, w2_ref[...], preferred_element_type=jnp.float32) + b2_ref[...]
    mean = jnp.mean(z, axis=0, keepdims=True)
    d = z - mean
    var = jnp.mean(d * d, axis=0, keepdims=True)
    zn = d * lax.rsqrt(var + BN_EPS) * g_ref[...] + bt_ref[...]
    zn = jnp.where(zn >= 0, zn, LEAKY_SLOPE * zn)
    out_ref[...] = h + zn


def kernel(h, edge_index, W1, b1, W2, b2, gamma, beta):
    src = edge_index[0].astype(jnp.int32)
    dst = edge_index[1].astype(jnp.int32)
    pad = E_PAD - N_EDGES
    src = jnp.concatenate([src, jnp.zeros((pad,), jnp.int32)])
    dst = jnp.concatenate([dst, jnp.full((pad,), N_NODES, jnp.int32)])
    src = src.reshape(NC, NS, CPW, CHUNK)
    dst = dst.reshape(NC, NS, CPW, CHUNK)
    zinit = jnp.zeros((N_PAD, D), jnp.float32)

    partials = _sc_scatter_sum(h, src, dst, zinit)
    p0 = partials[0, :N_NODES]
    p1 = partials[1, :N_NODES]

    out = pl.pallas_call(
        _tc_body,
        out_shape=jax.ShapeDtypeStruct((N_NODES, D), jnp.float32),
    )(h, p0, p1, W1, b1.reshape(1, D), W2, b2.reshape(1, D),
      gamma.reshape(1, D), beta.reshape(1, D))
    return out


# speedup vs baseline: 5.0292x; 5.0292x over previous
import jax
import jax.numpy as jnp
from jax import lax
from jax.experimental import pallas as pl
from jax.experimental.pallas import tpu as pltpu
from jax.experimental.pallas import tpu_sc as plsc

N_NODES = 10000
N_EDGES = 320000
D = 128
DH = D // 2
BN_EPS = 1e-5
LEAKY_SLOPE = 0.01

NC = 2
NS = 16
CHUNK = 128
CPB = 16
NBLK = 10
CPW = NBLK * CPB
E_PAD = NS * CPW * CHUNK
N_PAD = 10112
ROWS_PER_TILE = N_PAD // NS


def _sc_scatter_sum(h_split, src, dst, zinit):
    def body(h_hbm, src_hbm, dst_hbm, z_hbm, out_hbm,
             idx_s, idx_d, rows_a, rows_b, sem_a, sem_b, h_sh, aggr):
        c = lax.axis_index("c")
        s = lax.axis_index("s")
        r0 = s * ROWS_PER_TILE
        pltpu.sync_copy(h_hbm.at[c, pl.ds(r0, ROWS_PER_TILE)],
                        h_sh.at[pl.ds(r0, ROWS_PER_TILE)])
        pltpu.sync_copy(z_hbm.at[pl.ds(r0, ROWS_PER_TILE)],
                        aggr.at[pl.ds(r0, ROWS_PER_TILE)])
        plsc.subcore_barrier()

        def blk_body(b, carry0):
            b0 = pl.multiple_of(b * CPB, CPB)
            pltpu.sync_copy(src_hbm.at[s, pl.ds(b0, CPB)], idx_s)
            pltpu.sync_copy(dst_hbm.at[s, pl.ds(b0, CPB)], idx_d)
            pltpu.async_copy(h_sh.at[idx_s.at[0]], rows_a, sem_a)

            def step(k, carry):
                i = 2 * k
                j = i + 1
                pltpu.async_copy(h_sh.at[idx_s.at[j]], rows_b, sem_b)
                pltpu.make_async_copy(
                    h_sh.at[idx_s.at[i]], rows_a, sem_a).wait()
                pltpu.sync_copy(rows_a, aggr.at[idx_d.at[i]], add=True)

                @pl.when(k < (CPB // 2 - 1))
                def _():
                    pltpu.async_copy(h_sh.at[idx_s.at[j + 1]], rows_a, sem_a)

                pltpu.make_async_copy(
                    h_sh.at[idx_s.at[j]], rows_b, sem_b).wait()
                pltpu.sync_copy(rows_b, aggr.at[idx_d.at[j]], add=True)
                return carry

            lax.fori_loop(0, CPB // 2, step, 0)
            return carry0

        lax.fori_loop(0, NBLK, blk_body, 0)
        plsc.subcore_barrier()
        pltpu.sync_copy(aggr.at[pl.ds(r0, ROWS_PER_TILE)],
                        out_hbm.at[c, pl.ds(r0, ROWS_PER_TILE)])

    mesh = plsc.VectorSubcoreMesh(core_axis_name="c", subcore_axis_name="s")
    run = pl.kernel(
        body,
        out_type=jax.ShapeDtypeStruct((NC, N_PAD, DH), jnp.float32),
        mesh=mesh,
        compiler_params=pltpu.CompilerParams(use_tc_tiling_on_sc=False),
        scratch_types=[
            pltpu.VMEM((CPB, CHUNK), jnp.int32),
            pltpu.VMEM((CPB, CHUNK), jnp.int32),
            pltpu.VMEM((CHUNK, DH), jnp.float32),
            pltpu.VMEM((CHUNK, DH), jnp.float32),
            pltpu.SemaphoreType.DMA,
            pltpu.SemaphoreType.DMA,
            pltpu.VMEM_SHARED((N_PAD, DH), jnp.float32),
            pltpu.VMEM_SHARED((N_PAD, DH), jnp.float32),
        ],
    )
    return run(h_split, src, dst, zinit)


def _tc_body(h_ref, p_ref, w1_ref, b1_ref, w2_ref, b2_ref,
             g_ref, bt_ref, out_ref):
    h = h_ref[...]
    rst = h + p_ref[...]
    z = jnp.maximum(
        jnp.dot(rst, w1_ref[...], preferred_element_type=jnp.float32)
        + b1_ref[...], 0.0)
    z = jnp.dot(z, w2_ref[...], preferred_element_type=jnp.float32) + b2_ref[...]
    mean = jnp.mean(z, axis=0, keepdims=True)
    d = z - mean
    var = jnp.mean(d * d, axis=0, keepdims=True)
    zn = d * lax.rsqrt(var + BN_EPS) * g_ref[...] + bt_ref[...]
    zn = jnp.where(zn >= 0, zn, LEAKY_SLOPE * zn)
    out_ref[...] = h + zn


def kernel(h, edge_index, W1, b1, W2, b2, gamma, beta):
    src = edge_index[0].astype(jnp.int32)
    dst = edge_index[1].astype(jnp.int32)
    pad = E_PAD - N_EDGES
    src = jnp.concatenate([src, jnp.zeros((pad,), jnp.int32)])
    dst = jnp.concatenate([dst, jnp.full((pad,), N_NODES, jnp.int32)])
    src = src.reshape(NS, CPW, CHUNK)
    dst = dst.reshape(NS, CPW, CHUNK)
    h_split = jnp.zeros((NC, N_PAD, DH), jnp.float32)
    h_split = h_split.at[0, :N_NODES].set(h[:, :DH])
    h_split = h_split.at[1, :N_NODES].set(h[:, DH:])
    zinit = jnp.zeros((N_PAD, DH), jnp.float32)

    partials = _sc_scatter_sum(h_split, src, dst, zinit)
    p = jnp.concatenate([partials[0, :N_NODES], partials[1, :N_NODES]], axis=1)

    out = pl.pallas_call(
        _tc_body,
        out_shape=jax.ShapeDtypeStruct((N_NODES, D), jnp.float32),
    )(h, p, W1, b1.reshape(1, D), W2, b2.reshape(1, D),
      gamma.reshape(1, D), beta.reshape(1, D))
    return out


# speedup vs baseline: 5.4934x; 1.0923x over previous
import jax
import jax.numpy as jnp
from jax import lax
from jax.experimental import pallas as pl
from jax.experimental.pallas import tpu as pltpu
from jax.experimental.pallas import tpu_sc as plsc

N_NODES = 10000
N_EDGES = 320000
D = 128
DH = D // 2
BN_EPS = 1e-5
LEAKY_SLOPE = 0.01

NC = 2
NS = 16
CHUNK = 320
CPB = 8
NBLK = 8
CPW = NBLK * CPB
E_PAD = NS * CPW * CHUNK
N_PAD = 10112
ROWS_PER_TILE = N_PAD // NS


def _sc_scatter_sum(h_split, src, dst, zinit):
    def body(h_hbm, src_hbm, dst_hbm, z_hbm, out_hbm,
             idx_s, idx_d, rows_a, rows_b, sem_a, sem_b, h_sh, aggr):
        c = lax.axis_index("c")
        s = lax.axis_index("s")
        r0 = s * ROWS_PER_TILE
        pltpu.sync_copy(h_hbm.at[c, pl.ds(r0, ROWS_PER_TILE)],
                        h_sh.at[pl.ds(r0, ROWS_PER_TILE)])
        pltpu.sync_copy(z_hbm.at[pl.ds(r0, ROWS_PER_TILE)],
                        aggr.at[pl.ds(r0, ROWS_PER_TILE)])
        plsc.subcore_barrier()

        def blk_body(b, carry0):
            b0 = pl.multiple_of(b * CPB, CPB)
            pltpu.sync_copy(src_hbm.at[s, pl.ds(b0, CPB)], idx_s)
            pltpu.sync_copy(dst_hbm.at[s, pl.ds(b0, CPB)], idx_d)
            pltpu.async_copy(h_sh.at[idx_s.at[0]], rows_a, sem_a)

            def step(k, carry):
                i = 2 * k
                j = i + 1
                pltpu.async_copy(h_sh.at[idx_s.at[j]], rows_b, sem_b)
                pltpu.make_async_copy(
                    h_sh.at[idx_s.at[i]], rows_a, sem_a).wait()
                pltpu.sync_copy(rows_a, aggr.at[idx_d.at[i]], add=True)

                @pl.when(k < (CPB // 2 - 1))
                def _():
                    pltpu.async_copy(h_sh.at[idx_s.at[j + 1]], rows_a, sem_a)

                pltpu.make_async_copy(
                    h_sh.at[idx_s.at[j]], rows_b, sem_b).wait()
                pltpu.sync_copy(rows_b, aggr.at[idx_d.at[j]], add=True)
                return carry

            lax.fori_loop(0, CPB // 2, step, 0)
            return carry0

        lax.fori_loop(0, NBLK, blk_body, 0)
        plsc.subcore_barrier()
        pltpu.sync_copy(aggr.at[pl.ds(r0, ROWS_PER_TILE)],
                        out_hbm.at[c, pl.ds(r0, ROWS_PER_TILE)])

    mesh = plsc.VectorSubcoreMesh(core_axis_name="c", subcore_axis_name="s")
    run = pl.kernel(
        body,
        out_type=jax.ShapeDtypeStruct((NC, N_PAD, DH), jnp.float32),
        mesh=mesh,
        compiler_params=pltpu.CompilerParams(use_tc_tiling_on_sc=False),
        scratch_types=[
            pltpu.VMEM((CPB, CHUNK), jnp.int32),
            pltpu.VMEM((CPB, CHUNK), jnp.int32),
            pltpu.VMEM((CHUNK, DH), jnp.float32),
            pltpu.VMEM((CHUNK, DH), jnp.float32),
            pltpu.SemaphoreType.DMA,
            pltpu.SemaphoreType.DMA,
            pltpu.VMEM_SHARED((N_PAD, DH), jnp.float32),
            pltpu.VMEM_SHARED((N_PAD, DH), jnp.float32),
        ],
    )
    return run(h_split, src, dst, zinit)


def _tc_body(h_ref, p_ref, w1_ref, b1_ref, w2_ref, b2_ref,
             g_ref, bt_ref, out_ref):
    h = h_ref[...]
    rst = h + p_ref[...]
    z = jnp.maximum(
        jnp.dot(rst, w1_ref[...], preferred_element_type=jnp.float32)
        + b1_ref[...], 0.0)
    z = jnp.dot(z, w2_ref[...], preferred_element_type=jnp.float32) + b2_ref[...]
    mean = jnp.mean(z, axis=0, keepdims=True)
    d = z - mean
    var = jnp.mean(d * d, axis=0, keepdims=True)
    zn = d * lax.rsqrt(var + BN_EPS) * g_ref[...] + bt_ref[...]
    zn = jnp.where(zn >= 0, zn, LEAKY_SLOPE * zn)
    out_ref[...] = h + zn


def kernel(h, edge_index, W1, b1, W2, b2, gamma, beta):
    src = edge_index[0].astype(jnp.int32)
    dst = edge_index[1].astype(jnp.int32)
    pad = E_PAD - N_EDGES
    src = jnp.concatenate([src, jnp.zeros((pad,), jnp.int32)])
    dst = jnp.concatenate([dst, jnp.full((pad,), N_NODES, jnp.int32)])
    src = src.reshape(NS, CPW, CHUNK)
    dst = dst.reshape(NS, CPW, CHUNK)
    h_split = jnp.zeros((NC, N_PAD, DH), jnp.float32)
    h_split = h_split.at[0, :N_NODES].set(h[:, :DH])
    h_split = h_split.at[1, :N_NODES].set(h[:, DH:])
    zinit = jnp.zeros((N_PAD, DH), jnp.float32)

    partials = _sc_scatter_sum(h_split, src, dst, zinit)
    p = jnp.concatenate([partials[0, :N_NODES], partials[1, :N_NODES]], axis=1)

    out = pl.pallas_call(
        _tc_body,
        out_shape=jax.ShapeDtypeStruct((N_NODES, D), jnp.float32),
    )(h, p, W1, b1.reshape(1, D), W2, b2.reshape(1, D),
      gamma.reshape(1, D), beta.reshape(1, D))
    return out


# speedup vs baseline: 6.1017x; 1.1107x over previous
import jax
import jax.numpy as jnp
from jax import lax
from jax.experimental import pallas as pl
from jax.experimental.pallas import tpu as pltpu
from jax.experimental.pallas import tpu_sc as plsc

N_NODES = 10000
N_EDGES = 320000
D = 128
DH = D // 2
BN_EPS = 1e-5
LEAKY_SLOPE = 0.01

NC = 2
NS = 16
CHUNK = 320
CPB = 8
NBLK = 8
CPW = NBLK * CPB
E_PAD = NS * CPW * CHUNK
N_PAD = 10112
ROWS_PER_TILE = N_PAD // NS


LAST_ROWS = N_NODES - (NS - 1) * ROWS_PER_TILE


def _sc_scatter_sum(h, src, dst, zinit):
    def body(h_hbm, src_hbm, dst_hbm, z_hbm, out_hbm,
             idx_s, idx_d, rows_a, rows_b, sem_a, sem_b, h_sh, aggr):
        c = lax.axis_index("c")
        s = lax.axis_index("s")
        r0 = s * ROWS_PER_TILE
        c0 = c * DH

        @pl.when(s < NS - 1)
        def _():
            pltpu.sync_copy(h_hbm.at[pl.ds(r0, ROWS_PER_TILE), pl.ds(c0, DH)],
                            h_sh.at[pl.ds(r0, ROWS_PER_TILE)])

        @pl.when(s == NS - 1)
        def _():
            pltpu.sync_copy(
                h_hbm.at[pl.ds((NS - 1) * ROWS_PER_TILE, LAST_ROWS),
                         pl.ds(c0, DH)],
                h_sh.at[pl.ds((NS - 1) * ROWS_PER_TILE, LAST_ROWS)])

        pltpu.sync_copy(z_hbm.at[pl.ds(r0, ROWS_PER_TILE)],
                        aggr.at[pl.ds(r0, ROWS_PER_TILE)])
        plsc.subcore_barrier()

        def blk_body(b, carry0):
            b0 = pl.multiple_of(b * CPB, CPB)
            pltpu.sync_copy(src_hbm.at[s, pl.ds(b0, CPB)], idx_s)
            pltpu.sync_copy(dst_hbm.at[s, pl.ds(b0, CPB)], idx_d)
            pltpu.async_copy(h_sh.at[idx_s.at[0]], rows_a, sem_a)

            def step(k, carry):
                i = 2 * k
                j = i + 1
                pltpu.async_copy(h_sh.at[idx_s.at[j]], rows_b, sem_b)
                pltpu.make_async_copy(
                    h_sh.at[idx_s.at[i]], rows_a, sem_a).wait()
                pltpu.sync_copy(rows_a, aggr.at[idx_d.at[i]], add=True)

                @pl.when(k < (CPB // 2 - 1))
                def _():
                    pltpu.async_copy(h_sh.at[idx_s.at[j + 1]], rows_a, sem_a)

                pltpu.make_async_copy(
                    h_sh.at[idx_s.at[j]], rows_b, sem_b).wait()
                pltpu.sync_copy(rows_b, aggr.at[idx_d.at[j]], add=True)
                return carry

            lax.fori_loop(0, CPB // 2, step, 0)
            return carry0

        lax.fori_loop(0, NBLK, blk_body, 0)
        plsc.subcore_barrier()
        pltpu.sync_copy(aggr.at[pl.ds(r0, ROWS_PER_TILE)],
                        out_hbm.at[c, pl.ds(r0, ROWS_PER_TILE)])

    mesh = plsc.VectorSubcoreMesh(core_axis_name="c", subcore_axis_name="s")
    run = pl.kernel(
        body,
        out_type=jax.ShapeDtypeStruct((NC, N_PAD, DH), jnp.float32),
        mesh=mesh,
        compiler_params=pltpu.CompilerParams(use_tc_tiling_on_sc=False),
        scratch_types=[
            pltpu.VMEM((CPB, CHUNK), jnp.int32),
            pltpu.VMEM((CPB, CHUNK), jnp.int32),
            pltpu.VMEM((CHUNK, DH), jnp.float32),
            pltpu.VMEM((CHUNK, DH), jnp.float32),
            pltpu.SemaphoreType.DMA,
            pltpu.SemaphoreType.DMA,
            pltpu.VMEM_SHARED((N_PAD, DH), jnp.float32),
            pltpu.VMEM_SHARED((N_PAD, DH), jnp.float32),
        ],
    )
    return run(h, src, dst, zinit)


def _tc_body(h_ref, p_ref, w1_ref, b1_ref, w2_ref, b2_ref,
             g_ref, bt_ref, out_ref):
    h = h_ref[...]
    pf = p_ref[...]
    rst = h + jnp.concatenate([pf[0, :N_NODES], pf[1, :N_NODES]], axis=1)
    z = jnp.maximum(
        jnp.dot(rst, w1_ref[...], preferred_element_type=jnp.float32)
        + b1_ref[...], 0.0)
    z = jnp.dot(z, w2_ref[...], preferred_element_type=jnp.float32) + b2_ref[...]
    mean = jnp.mean(z, axis=0, keepdims=True)
    d = z - mean
    var = jnp.mean(d * d, axis=0, keepdims=True)
    zn = d * lax.rsqrt(var + BN_EPS) * g_ref[...] + bt_ref[...]
    zn = jnp.where(zn >= 0, zn, LEAKY_SLOPE * zn)
    out_ref[...] = h + zn


def kernel(h, edge_index, W1, b1, W2, b2, gamma, beta):
    src = edge_index[0].astype(jnp.int32)
    dst = edge_index[1].astype(jnp.int32)
    pad = E_PAD - N_EDGES
    src = jnp.concatenate([src, jnp.zeros((pad,), jnp.int32)])
    dst = jnp.concatenate([dst, jnp.full((pad,), N_NODES, jnp.int32)])
    src = src.reshape(NS, CPW, CHUNK)
    dst = dst.reshape(NS, CPW, CHUNK)
    zinit = jnp.zeros((N_PAD, DH), jnp.float32)

    partials = _sc_scatter_sum(h, src, dst, zinit)

    out = pl.pallas_call(
        _tc_body,
        out_shape=jax.ShapeDtypeStruct((N_NODES, D), jnp.float32),
    )(h, partials, W1, b1.reshape(1, D), W2, b2.reshape(1, D),
      gamma.reshape(1, D), beta.reshape(1, D))
    return out
